# baseline (speedup 1.0000x reference)
import jax
import jax.numpy as jnp
from jax.experimental import pallas as pl

_ROWS = 2048


def _ln_kernel(x_ref, pos_ref, gamma_ref, beta_ref, out_ref):
    emb = x_ref[0] + pos_ref[...]
    inv_e = 1.0 / emb.shape[-1]
    mean = jnp.sum(emb, axis=-1, keepdims=True) * inv_e
    ex2 = jnp.sum(emb * emb, axis=-1, keepdims=True) * inv_e
    var = ex2 - mean * mean
    scale = jax.lax.rsqrt(var + 1e-5)
    out_ref[0] = (emb - mean) * scale * gamma_ref[...] + beta_ref[...]


def kernel(x, pos_table, ln_gamma, ln_beta):
    B, S, E = x.shape
    gamma2 = ln_gamma.reshape(1, E)
    beta2 = ln_beta.reshape(1, E)
    grid = (S // _ROWS, B)
    return pl.pallas_call(
        _ln_kernel,
        grid=grid,
        in_specs=[
            pl.BlockSpec((1, _ROWS, E), lambda s, b: (b, s, 0)),
            pl.BlockSpec((_ROWS, E), lambda s, b: (s, 0)),
            pl.BlockSpec((1, E), lambda s, b: (0, 0)),
            pl.BlockSpec((1, E), lambda s, b: (0, 0)),
        ],
        out_specs=pl.BlockSpec((1, _ROWS, E), lambda s, b: (b, s, 0)),
        out_shape=jax.ShapeDtypeStruct((B, S, E), x.dtype),
    )(x, pos_table, gamma2, beta2)


# speedup vs baseline: 1.0011x; 1.0011x over previous
import jax
import jax.numpy as jnp
from jax.experimental import pallas as pl
from jax.experimental.pallas import tpu as pltpu

_ROWS = 2048


def _ln_kernel(x_ref, pos_ref, gamma_ref, beta_ref, out_ref):
    emb = x_ref[0] + pos_ref[...]
    inv_e = 1.0 / emb.shape[-1]
    mean = jnp.sum(emb, axis=-1, keepdims=True) * inv_e
    ex2 = jnp.sum(emb * emb, axis=-1, keepdims=True) * inv_e
    var = ex2 - mean * mean
    scale = jax.lax.rsqrt(var + 1e-5)
    out_ref[0] = (emb - mean) * scale * gamma_ref[...] + beta_ref[...]


def kernel(x, pos_table, ln_gamma, ln_beta):
    B, S, E = x.shape
    gamma2 = ln_gamma.reshape(1, E)
    beta2 = ln_beta.reshape(1, E)
    grid = (S // _ROWS, B)
    return pl.pallas_call(
        _ln_kernel,
        grid=grid,
        in_specs=[
            pl.BlockSpec((1, _ROWS, E), lambda s, b: (b, s, 0)),
            pl.BlockSpec((_ROWS, E), lambda s, b: (s, 0)),
            pl.BlockSpec((1, E), lambda s, b: (0, 0)),
            pl.BlockSpec((1, E), lambda s, b: (0, 0)),
        ],
        out_specs=pl.BlockSpec((1, _ROWS, E), lambda s, b: (b, s, 0)),
        out_shape=jax.ShapeDtypeStruct((B, S, E), x.dtype),
        compiler_params=pltpu.CompilerParams(
            dimension_semantics=("parallel", "parallel"),
        ),
    )(x, pos_table, gamma2, beta2)
